# baseline (speedup 1.0000x reference)
import jax
import jax.numpy as jnp
import numpy as np
from jax import lax
from jax.experimental import pallas as pl
from jax.experimental.pallas import tpu as pltpu
from jax.experimental.pallas import tpu_sc as plsc

_B = 16
_N = 4096
_S = 512
_K = 32
_C = 64
_R2 = np.float32(0.2 * 0.2)
_P = _B * _S * _K
_ROWS = 2048
_GRID = _P // _ROWS
_CEN = _ROWS // _K


def _fps_body(xs_ref, ys_ref, zs_ref, cx_ref, cy_ref, cz_ref, dist_ref):
    iota = lax.broadcasted_iota(jnp.int32, (_B, _N), 1)
    iota_s = lax.broadcasted_iota(jnp.int32, (_B, _S), 1)
    dist_ref[...] = jnp.full((_B, _N), 1e10, jnp.float32)
    xs = xs_ref[...]
    ys = ys_ref[...]
    zs = zs_ref[...]

    def step(i, carry):
        cx, cy, cz, ax, ay, az = carry
        sel_o = iota_s == i
        ax = jnp.where(sel_o, cx, ax)
        ay = jnp.where(sel_o, cy, ay)
        az = jnp.where(sel_o, cz, az)
        dx = xs - cx
        dy = ys - cy
        dz = zs - cz
        d = dx * dx + dy * dy + dz * dz
        dist = jnp.minimum(dist_ref[...], d)
        dist_ref[...] = dist
        m = jnp.max(dist, axis=1, keepdims=True)
        sel = dist == m
        cx = jnp.max(jnp.where(sel, xs, -1e30), axis=1, keepdims=True)
        cy = jnp.max(jnp.where(sel, ys, -1e30), axis=1, keepdims=True)
        cz = jnp.max(jnp.where(sel, zs, -1e30), axis=1, keepdims=True)
        return cx, cy, cz, ax, ay, az

    zc = jnp.zeros((_B, _S), jnp.float32)
    _, _, _, ax, ay, az = lax.fori_loop(
        0, _S, step,
        (xs[:, 0:1], ys[:, 0:1], zs[:, 0:1], zc, zc, zc))
    cx_ref[...] = ax
    cy_ref[...] = ay
    cz_ref[...] = az


def _fps(xs, ys, zs):
    return pl.pallas_call(
        _fps_body,
        out_shape=[jax.ShapeDtypeStruct((_B, _S), jnp.float32)] * 3,
        scratch_shapes=[pltpu.VMEM((_B, _N), jnp.float32)],
    )(xs, ys, zs)


def _table2_body(pts_ref, xyz8_ref, w0f_ref, w0x_ref, b0_ref, out_ref):
    t = jnp.dot(pts_ref[0].astype(jnp.bfloat16), w0f_ref[...].astype(jnp.bfloat16),
                preferred_element_type=jnp.float32)
    t += jnp.dot(xyz8_ref[0].astype(jnp.bfloat16), w0x_ref[...].astype(jnp.bfloat16),
                 preferred_element_type=jnp.float32)
    out_ref[0] = t + b0_ref[0:1, :_C]


def _table2(pts_t, xyz8, w0f, w0x, b0):
    return pl.pallas_call(
        _table2_body,
        grid=(_B,),
        in_specs=[
            pl.BlockSpec((1, _N, _C), lambda b: (b, 0, 0)),
            pl.BlockSpec((1, _N, 8), lambda b: (b, 0, 0)),
            pl.BlockSpec((_C, _C), lambda b: (0, 0)),
            pl.BlockSpec((8, _C), lambda b: (0, 0)),
            pl.BlockSpec((8, 128), lambda b: (0, 0)),
        ],
        out_specs=pl.BlockSpec((1, _N, _C), lambda b: (b, 0, 0)),
        out_shape=jax.ShapeDtypeStruct((_B, _N, _C), jnp.float32),
        compiler_params=pltpu.CompilerParams(dimension_semantics=("arbitrary",)),
    )(pts_t, xyz8, w0f, w0x, b0)


_SH = _S // 2
_CHUNK = 256
_BUF = _K - 1 + _CHUNK + 16


_QUAD = 4
_NQ = _SH // _QUAD
_D = 4


def _sc_body(xs_h, ys_h, zs_h, cx_h, cy_h, cz_h, tab_h, feat_h,
             x_v, y_v, z_v, cx_v, cy_v, cz_v, idxb, idxring, featring,
             semg, semo):
    wid = lax.axis_index("s") * 2 + lax.axis_index("c")
    b = wid // 2
    s0 = (wid % 2) * _SH
    pltpu.sync_copy(xs_h.at[b], x_v)
    pltpu.sync_copy(ys_h.at[b], y_v)
    pltpu.sync_copy(zs_h.at[b], z_v)
    pltpu.sync_copy(cx_h.at[b, pl.ds(s0, _SH)], cx_v.at[pl.ds(0, _SH)])
    pltpu.sync_copy(cy_h.at[b, pl.ds(s0, _SH)], cy_v.at[pl.ds(0, _SH)])
    pltpu.sync_copy(cz_h.at[b, pl.ds(s0, _SH)], cz_v.at[pl.ds(0, _SH)])

    lane = lax.iota(jnp.int32, 16)
    _CLAMP = _BUF - 16

    def ball_query4(q, slot):
        sl0 = q * _QUAD
        cen = [(cx_v[pl.ds(sl0 + j, 16)][0],
                cy_v[pl.ds(sl0 + j, 16)][0],
                cz_v[pl.ds(sl0 + j, 16)][0]) for j in range(_QUAD)]

        def big_chunk(g, cnts):
            def subchunk(u, cs):
                n0 = g * _CHUNK + u * 16
                xv = x_v[pl.ds(n0, 16)]
                yv = y_v[pl.ds(n0, 16)]
                zv = z_v[pl.ds(n0, 16)]
                out = []
                for j in range(_QUAD):
                    cxs, cys, czs = cen[j]
                    dx = xv - cxs
                    dy = yv - cys
                    dz = zv - czs
                    d = dx * dx + dy * dy + dz * dz
                    mi = jnp.where(d <= _R2, 1, 0)
                    incl = mi
                    for k in (1, 2, 4, 8):
                        incl = incl + jnp.where(
                            lane >= k, incl[jnp.maximum(lane - k, 0)], 0)
                    lo = jnp.zeros((16,), jnp.int32)
                    for step in (8, 4, 2, 1):
                        t = incl[lo + (step - 1)]
                        lo = jnp.where(t <= lane, lo + step, lo)
                    c = cs[j]
                    idxb[j, pl.ds(jnp.minimum(c, _CLAMP), 16)] = n0 + lo
                    out.append(c + incl[15])
                return tuple(out)

            def scan256(cs):
                return lax.fori_loop(0, _CHUNK // 16, subchunk, cs)

            mn = jnp.minimum(jnp.minimum(cnts[0], cnts[1]),
                             jnp.minimum(cnts[2], cnts[3]))
            return lax.cond(mn < _K, scan256, lambda cs: cs, cnts)

        zero = jnp.int32(0)
        cnts = lax.fori_loop(0, _N // _CHUNK, big_chunk,
                             (zero, zero, zero, zero))

        for j in range(_QUAD):
            i0 = idxb[j, pl.ds(0, 16)][0]
            for h in range(2):
                lanepos = h * 16 + lane
                idxring[slot, pl.ds(j * _K + h * 16, 16)] = jnp.where(
                    lanepos < cnts[j], idxb[j, pl.ds(h * 16, 16)], i0)

    def out_slice(q):
        p0 = (b * _S + s0 + q * _QUAD) * _K
        return feat_h.at[pl.ds(p0, _QUAD * _K), :]

    def per_quad(q, carry):
        slot = lax.rem(q, _D)

        @pl.when(q >= _D)
        def _():
            pltpu.make_async_copy(
                featring.at[slot], out_slice(q), semo.at[slot]).wait()

        ball_query4(q, slot)

        pltpu.async_copy(tab_h.at[b].at[idxring.at[slot]], featring.at[slot],
                         semg.at[slot])

        @pl.when(q >= 1)
        def _():
            slot2 = lax.rem(q - 1, _D)
            pltpu.make_async_copy(
                tab_h.at[b, pl.ds(0, _QUAD * _K), :], featring.at[slot2],
                semg.at[slot2]).wait()
            pltpu.async_copy(featring.at[slot2], out_slice(q - 1),
                             semo.at[slot2])

        return carry

    lax.fori_loop(0, _NQ, per_quad, jnp.int32(0))

    lastslot = (_NQ - 1) % _D
    pltpu.make_async_copy(tab_h.at[b, pl.ds(0, _QUAD * _K), :],
                          featring.at[lastslot], semg.at[lastslot]).wait()
    pltpu.async_copy(featring.at[lastslot], out_slice(_NQ - 1),
                     semo.at[lastslot])
    for k in range(_D):
        pltpu.make_async_copy(featring.at[k], out_slice(0),
                              semo.at[k]).wait()


def _ball_gather(xs, ys, zs, cx, cy, cz, tab):
    kern = pl.kernel(
        _sc_body,
        out_type=[jax.ShapeDtypeStruct((_P, _C), jnp.float32)],
        mesh=plsc.VectorSubcoreMesh(
            core_axis_name="c", subcore_axis_name="s",
            num_cores=2, num_subcores=16),
        compiler_params=pltpu.CompilerParams(use_tc_tiling_on_sc=False),
        scratch_types=[
            pltpu.VMEM((_N,), jnp.float32),
            pltpu.VMEM((_N,), jnp.float32),
            pltpu.VMEM((_N,), jnp.float32),
            pltpu.VMEM((_SH + 16,), jnp.float32),
            pltpu.VMEM((_SH + 16,), jnp.float32),
            pltpu.VMEM((_SH + 16,), jnp.float32),
            pltpu.VMEM((_QUAD, _BUF), jnp.int32),
            pltpu.VMEM((_D, _QUAD * _K), jnp.int32),
            pltpu.VMEM((_D, _QUAD * _K, _C), jnp.float32),
            pltpu.SemaphoreType.DMA((_D,)),
            pltpu.SemaphoreType.DMA((_D,)),
        ],
    )
    (feat,) = kern(xs, ys, zs, cx, cy, cz, tab)
    return feat


def _bf(x):
    return x.astype(jnp.bfloat16)


def _y0_tile(g_ref, cen8_ref, w0x_ref):
    vt = jnp.dot(_bf(cen8_ref[...]), _bf(w0x_ref[...]),
                 preferred_element_type=jnp.float32)
    vt = jnp.broadcast_to(vt[:, None, :], (_CEN, _K, _C)).reshape(_ROWS, _C)
    return g_ref[...] - vt


def _next_layer(z, w_ref, b_ref, co):
    y = jnp.dot(_bf(z), _bf(w_ref[...]), preferred_element_type=jnp.float32)
    return y + b_ref[0:1, :co]


def _apply_bn_relu(y, a_ref, c_ref, co):
    return jnp.maximum(y * a_ref[0:1, :co] + c_ref[0:1, :co], 0.0)


def _acc_stats(y, co, sum_ref, ssq_ref):
    @pl.when(pl.program_id(0) == 0)
    def _():
        sum_ref[...] = jnp.zeros_like(sum_ref)
        ssq_ref[...] = jnp.zeros_like(ssq_ref)

    sum_ref[0:1, :co] += jnp.sum(y, axis=0, keepdims=True)
    ssq_ref[0:1, :co] += jnp.sum(y * y, axis=0, keepdims=True)


def _sweep1_body(g, cen8, w0x, sum_ref, ssq_ref):
    y0 = _y0_tile(g, cen8, w0x)
    _acc_stats(y0, _C, sum_ref, ssq_ref)


def _sweep2_body(g, cen8, w0x, a0, c0, w1, b1, sum_ref, ssq_ref):
    y0 = _y0_tile(g, cen8, w0x)
    z0 = _apply_bn_relu(y0, a0, c0, _C)
    y1 = _next_layer(z0, w1, b1, _C)
    _acc_stats(y1, _C, sum_ref, ssq_ref)


def _sweep3_body(g, cen8, w0x, a0, c0, w1, b1, a1, c1, w2, b2,
                 sum_ref, ssq_ref):
    y0 = _y0_tile(g, cen8, w0x)
    z0 = _apply_bn_relu(y0, a0, c0, _C)
    y1 = _next_layer(z0, w1, b1, _C)
    z1 = _apply_bn_relu(y1, a1, c1, _C)
    y2 = _next_layer(z1, w2, b2, 128)
    _acc_stats(y2, 128, sum_ref, ssq_ref)


def _sweep4_body(g, cen8, w0x, a0, c0, w1, b1, a1, c1, w2, b2, a2, c2,
                 out_ref):
    y0 = _y0_tile(g, cen8, w0x)
    z0 = _apply_bn_relu(y0, a0, c0, _C)
    y1 = _next_layer(z0, w1, b1, _C)
    z1 = _apply_bn_relu(y1, a1, c1, _C)
    y2 = _next_layer(z1, w2, b2, 128)
    z2 = _apply_bn_relu(y2, a2, c2, 128)
    out_ref[...] = jnp.max(z2.reshape(_CEN, _K, 128), axis=1)


def _const_spec(shape):
    return pl.BlockSpec(shape, lambda i: (0, 0))


_G_SPEC = pl.BlockSpec((_ROWS, _C), lambda i: (i, 0))
_CEN_SPEC = pl.BlockSpec((_CEN, 8), lambda i: (i, 0))
_STATS_SPEC = pl.BlockSpec((8, 128), lambda i: (0, 0))
_STATS_SHAPE = jax.ShapeDtypeStruct((8, 128), jnp.float32)
_CPARAMS = pltpu.CompilerParams(dimension_semantics=("arbitrary",))


def _sweep(body, small_shapes, out_shapes, out_specs):
    return pl.pallas_call(
        body,
        grid=(_GRID,),
        in_specs=[_G_SPEC, _CEN_SPEC] + [_const_spec(s) for s in small_shapes],
        out_specs=out_specs,
        out_shape=out_shapes,
        compiler_params=_CPARAMS,
    )


def _stats_to_affine(stats, gamma, beta, co):
    sum_o, ssq_o = stats
    s = sum_o[0, :co]
    q = ssq_o[0, :co]
    mean = s / _P
    var = q / _P - mean * mean
    a = gamma / jnp.sqrt(var + 1e-5)
    c = beta - mean * a
    return _pad_row(a), _pad_row(c)


def _pad_row(v):
    out = jnp.zeros((8, 128), jnp.float32)
    return out.at[0, :v.shape[0]].set(v)


def kernel(xyz, points, params):
    xyz_t = jnp.transpose(xyz, (2, 0, 1))
    xs, ys, zs = xyz_t[0], xyz_t[1], xyz_t[2]
    pts_t = jnp.transpose(points, (0, 2, 1))
    xyz8 = jnp.pad(xyz, ((0, 0), (0, 0), (0, 5)))

    w0 = params["W0"]
    w0f = jnp.transpose(w0[:, :_C])
    w0x = jnp.zeros((8, _C), jnp.float32).at[:3].set(jnp.transpose(w0[:, _C:]))
    b0 = _pad_row(params["b0"])
    w1 = jnp.transpose(params["W1"])
    b1 = _pad_row(params["b1"])
    w2 = jnp.transpose(params["W2"])
    b2 = _pad_row(params["b2"])

    cx, cy, cz = _fps(xs, ys, zs)
    new_xyz = jnp.stack([cx, cy, cz], axis=-1)
    cen8 = jnp.pad(new_xyz.reshape(_B * _S, 3), ((0, 0), (0, 5)))

    tab = _table2(pts_t, xyz8, w0f, w0x, b0)
    g = _ball_gather(xs, ys, zs, cx, cy, cz, tab)

    stats0 = _sweep(
        _sweep1_body, [(8, _C)],
        [_STATS_SHAPE] * 2, [_STATS_SPEC] * 2,
    )(g, cen8, w0x)
    a0, c0 = _stats_to_affine(stats0, params["gamma0"], params["beta0"], _C)

    stats1 = _sweep(
        _sweep2_body, [(8, _C), (8, 128), (8, 128), (64, _C), (8, 128)],
        [_STATS_SHAPE] * 2, [_STATS_SPEC] * 2,
    )(g, cen8, w0x, a0, c0, w1, b1)
    a1, c1 = _stats_to_affine(stats1, params["gamma1"], params["beta1"], _C)

    stats2 = _sweep(
        _sweep3_body,
        [(8, _C), (8, 128), (8, 128), (64, _C), (8, 128),
         (8, 128), (8, 128), (64, 128), (8, 128)],
        [_STATS_SHAPE] * 2, [_STATS_SPEC] * 2,
    )(g, cen8, w0x, a0, c0, w1, b1, a1, c1, w2, b2)
    a2, c2 = _stats_to_affine(stats2, params["gamma2"], params["beta2"], 128)

    pooled = _sweep(
        _sweep4_body,
        [(8, _C), (8, 128), (8, 128), (64, _C), (8, 128),
         (8, 128), (8, 128), (64, 128), (8, 128), (8, 128), (8, 128)],
        jax.ShapeDtypeStruct((_P // _K, 128), jnp.float32),
        pl.BlockSpec((_CEN, 128), lambda i: (i, 0)),
    )(g, cen8, w0x, a0, c0, w1, b1, a1, c1, w2, b2, a2, c2)

    new_points = jnp.transpose(pooled.reshape(_B, _S, 128), (0, 2, 1))
    return new_xyz, new_points
